# baseline (speedup 1.0000x reference)
import jax
import jax.numpy as jnp
from jax import lax
from jax.experimental import pallas as pl
from jax.experimental.pallas import tpu as pltpu
from jax.experimental.pallas import tpu_sc as plsc

NUM_CORES = 2
NUM_SUBCORES = 16
LANES = 16
NW = NUM_CORES * NUM_SUBCORES
IDX_CHUNK = 128


def _mf_body(u_hbm, it_hbm, a_hbm, b_hbm, bu_hbm, bi_hbm, mu_hbm, out_hbm,
             u_v, it_v, av, bv, buv, biv, mu_v, out_v,
             sem_a, sem_b, sem_bu, sem_bi):
    bpw = out_v.shape[0]
    n_chunks = u_v.shape[0]
    r = av.shape[1]
    wid = lax.axis_index("s") * NUM_CORES + lax.axis_index("c")
    base = wid * bpw

    pltpu.sync_copy(u_hbm.at[pl.ds(wid * n_chunks, n_chunks)], u_v)
    pltpu.sync_copy(it_hbm.at[pl.ds(wid * n_chunks, n_chunks)], it_v)
    pltpu.sync_copy(mu_hbm, mu_v)

    copies = []
    for k in range(n_chunks):
        dst = pl.ds(k * IDX_CHUNK, IDX_CHUNK)
        copies.append(pltpu.async_copy(a_hbm.at[u_v.at[k]], av.at[dst], sem_a))
        copies.append(pltpu.async_copy(b_hbm.at[it_v.at[k]], bv.at[dst], sem_b))
        copies.append(pltpu.async_copy(bu_hbm.at[u_v.at[k]], buv.at[dst], sem_bu))
        copies.append(pltpu.async_copy(bi_hbm.at[it_v.at[k]], biv.at[dst], sem_bi))
    for c in copies:
        c.wait()

    iota = lax.iota(jnp.int32, LANES)

    def group(g, carry):
        rows = g * LANES + iota
        acc = mu_v[...] + buv[pl.ds(g * LANES, LANES)] + biv[pl.ds(g * LANES, LANES)]
        for j in range(r):
            cols = jnp.full((LANES,), j, jnp.int32)
            a = plsc.load_gather(av, [rows, cols])
            b = plsc.load_gather(bv, [rows, cols])
            acc = acc + a * b
        out_v[pl.ds(g * LANES, LANES)] = acc
        return carry

    lax.fori_loop(0, bpw // LANES, group, 0)
    pltpu.sync_copy(out_v, out_hbm.at[pl.ds(base, bpw)])


def kernel(u, it, A, B, bu, bi, mu):
    batch = u.shape[0]
    r = A.shape[1]
    bpw = batch // NW
    n_chunks = bpw // IDX_CHUNK
    u2 = u.astype(jnp.int32).reshape(NW * n_chunks, IDX_CHUNK)
    it2 = it.astype(jnp.int32).reshape(NW * n_chunks, IDX_CHUNK)
    mu16 = jnp.broadcast_to(jnp.asarray(mu, jnp.float32), (LANES,))

    mesh = plsc.VectorSubcoreMesh(core_axis_name="c", subcore_axis_name="s")
    f = pl.kernel(
        _mf_body,
        out_type=jax.ShapeDtypeStruct((batch,), jnp.float32),
        mesh=mesh,
        compiler_params=pltpu.CompilerParams(
            needs_layout_passes=False, use_tc_tiling_on_sc=False
        ),
        scratch_types=[
            pltpu.VMEM((n_chunks, IDX_CHUNK), jnp.int32),
            pltpu.VMEM((n_chunks, IDX_CHUNK), jnp.int32),
            pltpu.VMEM((bpw, r), jnp.float32),
            pltpu.VMEM((bpw, r), jnp.float32),
            pltpu.VMEM((bpw,), jnp.float32),
            pltpu.VMEM((bpw,), jnp.float32),
            pltpu.VMEM((LANES,), jnp.float32),
            pltpu.VMEM((bpw,), jnp.float32),
            pltpu.SemaphoreType.DMA,
            pltpu.SemaphoreType.DMA,
            pltpu.SemaphoreType.DMA,
            pltpu.SemaphoreType.DMA,
        ],
    )
    return f(u2, it2, A, B, bu, bi, mu16)


# speedup vs baseline: 1.5053x; 1.5053x over previous
import jax
import jax.numpy as jnp
from jax import lax
from jax.experimental import pallas as pl
from jax.experimental.pallas import tpu as pltpu
from jax.experimental.pallas import tpu_sc as plsc

NUM_CORES = 2
NUM_SUBCORES = 16
LANES = 16
NW = NUM_CORES * NUM_SUBCORES
IDX_CHUNK = 128
TILE_ROWS = 8
CHUNK = 32


def _bias_body(u_hbm, it_hbm, bu_hbm, bi_hbm, mu_hbm, out_hbm,
               u_v, it_v, buv, biv, mu_v, out_v, sem_bu, sem_bi):
    bpw = out_v.shape[0]
    n_chunks = u_v.shape[0]
    wid = lax.axis_index("s") * NUM_CORES + lax.axis_index("c")
    base = wid * bpw

    pltpu.sync_copy(u_hbm.at[pl.ds(wid * n_chunks, n_chunks)], u_v)
    pltpu.sync_copy(it_hbm.at[pl.ds(wid * n_chunks, n_chunks)], it_v)
    pltpu.sync_copy(mu_hbm, mu_v)

    copies = []
    for k in range(n_chunks):
        dst = pl.ds(k * IDX_CHUNK, IDX_CHUNK)
        copies.append(pltpu.async_copy(bu_hbm.at[u_v.at[k]], buv.at[dst], sem_bu))
        copies.append(pltpu.async_copy(bi_hbm.at[it_v.at[k]], biv.at[dst], sem_bi))
    for c in copies:
        c.wait()

    def group(g, carry):
        s = pl.ds(g * LANES, LANES)
        out_v[s] = mu_v[...] + buv[s] + biv[s]
        return carry

    lax.fori_loop(0, bpw // LANES, group, 0)
    pltpu.sync_copy(out_v, out_hbm.at[pl.ds(base, bpw)])


def _factor_body(u_hbm, it_hbm, a_hbm, b_hbm, bs_hbm, out_hbm,
                 u_v, it_v, bs_v, ta, tb, out_v, sem_a, sem_b):
    bpw = out_v.shape[0]
    r = a_hbm.shape[1]
    wid = lax.axis_index("s") * NUM_CORES + lax.axis_index("c")
    base = wid * bpw

    pltpu.sync_copy(u_hbm.at[pl.ds(base, bpw)], u_v)
    pltpu.sync_copy(it_hbm.at[pl.ds(base, bpw)], it_v)
    pltpu.sync_copy(bs_hbm.at[pl.ds(base, bpw)], bs_v)

    iota = lax.iota(jnp.int32, LANES)
    n_groups = CHUNK // LANES

    def chunk_body(c, carry):
        uvecs = [u_v[pl.ds(c * CHUNK + h * LANES, LANES)] for h in range(n_groups)]
        ivecs = [it_v[pl.ds(c * CHUNK + h * LANES, LANES)] for h in range(n_groups)]
        for h in range(n_groups):
            for l in range(LANES):
                slot = h * LANES + l
                tu = (uvecs[h][l] >> 3) * TILE_ROWS
                ti = (ivecs[h][l] >> 3) * TILE_ROWS
                pltpu.async_copy(a_hbm.at[pl.ds(tu, TILE_ROWS)], ta.at[slot], sem_a)
                pltpu.async_copy(b_hbm.at[pl.ds(ti, TILE_ROWS)], tb.at[slot], sem_b)
        for slot in range(CHUNK):
            pltpu.make_async_copy(
                a_hbm.at[pl.ds(0, TILE_ROWS)], ta.at[slot], sem_a).wait()
            pltpu.make_async_copy(
                b_hbm.at[pl.ds(0, TILE_ROWS)], tb.at[slot], sem_b).wait()
        for h in range(n_groups):
            acc = bs_v[pl.ds(c * CHUNK + h * LANES, LANES)]
            for l in range(LANES):
                slot = h * LANES + l
                su = uvecs[h][l] & (TILE_ROWS - 1)
                si = ivecs[h][l] & (TILE_ROWS - 1)
                ra = ta.at[slot].at[su]
                rb = tb.at[slot].at[si]
                q = ra[pl.ds(0, LANES)] * rb[pl.ds(0, LANES)]
                for k in range(1, r // LANES):
                    q = q + ra[pl.ds(k * LANES, LANES)] * rb[pl.ds(k * LANES, LANES)]
                s = jnp.sum(q, axis=0)
                acc = jnp.where(iota == l, acc + s, acc)
            out_v[pl.ds(c * CHUNK + h * LANES, LANES)] = acc
        return carry

    lax.fori_loop(0, bpw // CHUNK, chunk_body, 0)
    pltpu.sync_copy(out_v, out_hbm.at[pl.ds(base, bpw)])


def kernel(u, it, A, B, bu, bi, mu):
    batch = u.shape[0]
    r = A.shape[1]
    bpw = batch // NW
    n_chunks = bpw // IDX_CHUNK
    u1 = u.astype(jnp.int32)
    it1 = it.astype(jnp.int32)
    u2 = u1.reshape(NW * n_chunks, IDX_CHUNK)
    it2 = it1.reshape(NW * n_chunks, IDX_CHUNK)
    mu16 = jnp.broadcast_to(jnp.asarray(mu, jnp.float32), (LANES,))

    mesh = plsc.VectorSubcoreMesh(core_axis_name="c", subcore_axis_name="s")
    bias_f = pl.kernel(
        _bias_body,
        out_type=jax.ShapeDtypeStruct((batch,), jnp.float32),
        mesh=mesh,
        compiler_params=pltpu.CompilerParams(
            needs_layout_passes=False, use_tc_tiling_on_sc=False
        ),
        scratch_types=[
            pltpu.VMEM((n_chunks, IDX_CHUNK), jnp.int32),
            pltpu.VMEM((n_chunks, IDX_CHUNK), jnp.int32),
            pltpu.VMEM((bpw,), jnp.float32),
            pltpu.VMEM((bpw,), jnp.float32),
            pltpu.VMEM((LANES,), jnp.float32),
            pltpu.VMEM((bpw,), jnp.float32),
            pltpu.SemaphoreType.DMA,
            pltpu.SemaphoreType.DMA,
        ],
    )
    bs = bias_f(u2, it2, bu, bi, mu16)

    factor_f = pl.kernel(
        _factor_body,
        out_type=jax.ShapeDtypeStruct((batch,), jnp.float32),
        mesh=mesh,
        compiler_params=pltpu.CompilerParams(
            needs_layout_passes=False, use_tc_tiling_on_sc=True
        ),
        scratch_types=[
            pltpu.VMEM((bpw,), jnp.int32),
            pltpu.VMEM((bpw,), jnp.int32),
            pltpu.VMEM((bpw,), jnp.float32),
            pltpu.VMEM((CHUNK, TILE_ROWS, r), jnp.float32),
            pltpu.VMEM((CHUNK, TILE_ROWS, r), jnp.float32),
            pltpu.VMEM((bpw,), jnp.float32),
            pltpu.SemaphoreType.DMA,
            pltpu.SemaphoreType.DMA,
        ],
    )
    return factor_f(u1, it1, A, B, bs)
